# baseline (speedup 1.0000x reference)
import math

import jax
import jax.numpy as jnp
from jax.experimental import pallas as pl
from jax.experimental.pallas import tpu as pltpu


def _sr_kernel(z_ref, cb_ref, cbt_ref, cbbt_ref, wr_ref, br_ref,
               idx_ref, mask_ref, out_ref,
               msmt_ref, cbsq_ref, tbl_ref, s_ref, cnt_fix_ref):
    b = pl.program_id(0)
    nb = pl.num_programs(0)
    L = z_ref.shape[1]
    K = cb_ref.shape[0]
    R = wr_ref.shape[0]

    @pl.when(b == 0)
    def _init():
        cwr = jax.lax.dot_general(
            cb_ref[...], wr_ref[...], (((1,), (1,)), ((), ())),
            preferred_element_type=jnp.float32)
        logits = cwr + br_ref[0][None, :]
        mx = jnp.max(logits, axis=1, keepdims=True)
        e = jnp.exp(logits - mx)
        msm = e / jnp.sum(e, axis=1, keepdims=True)
        msmt = msm.T
        msmt_ref[...] = msmt
        cbsq_ref[...] = jnp.sum(cbt_ref[...] * cbt_ref[...], axis=0,
                                keepdims=True)
        tbl_ref[0:R, :] = msmt.astype(jnp.bfloat16)
        r_i = jax.lax.broadcasted_iota(jnp.int32, (8, K), 0)
        k_i = jax.lax.broadcasted_iota(jnp.int32, (8, K), 1)
        w = jnp.where(r_i == 0, 1,
                      jnp.where(r_i == 1, k_i // 16,
                                jnp.where(r_i == 2, k_i % 16, 0)))
        tbl_ref[R:R + 8, :] = w.astype(jnp.bfloat16)

    @pl.when(b < nb - 1)
    def _batch():
        NCH = 4
        CL = L // NCH
        counts_acc = None
        den_acc = None
        for ci in range(NCH):
            z = z_ref[0, ci * CL:(ci + 1) * CL, :]
            zsq = jnp.sum(z * z, axis=1, keepdims=True)
            zc = jax.lax.dot_general(
                z.astype(jnp.bfloat16), cbbt_ref[...],
                (((1,), (0,)), ((), ())),
                preferred_element_type=jnp.float32)
            d = zsq - 2.0 * zc + cbsq_ref[...]
            dmin = jnp.min(d, axis=1, keepdims=True)
            mtie = d == dmin
            mtb = mtie.astype(jnp.bfloat16)
            res = jax.lax.dot_general(
                tbl_ref[...], mtb, (((1,), (1,)), ((), ())),
                preferred_element_type=jnp.float32)
            counts = jax.lax.dot_general(
                jnp.ones((1, CL), jnp.bfloat16), mtb,
                (((1,), (0,)), ((), ())),
                preferred_element_type=jnp.float32)
            tc = res[R:R + 1]
            idx_c = (res[R + 1:R + 2] * 16.0
                     + res[R + 2:R + 3]).astype(jnp.int32)
            mask_c = res[0:R]
            idx_ref[0, :, ci * CL:(ci + 1) * CL] = idx_c
            mask_ref[0, :, ci * CL:(ci + 1) * CL] = mask_c

            @pl.when(jnp.max(tc) > 1.5)
            def _tie_fix(mtie=mtie, ci=ci):
                iota_l = jax.lax.broadcasted_iota(jnp.int32, (CL, K), 1)
                strict = jnp.min(jnp.where(mtie, iota_l, K), axis=1,
                                 keepdims=True)
                ohb = (iota_l == strict).astype(jnp.bfloat16)
                res_s = jax.lax.dot_general(
                    tbl_ref[...], ohb, (((1,), (1,)), ((), ())),
                    preferred_element_type=jnp.float32)
                counts_s = jax.lax.dot_general(
                    jnp.ones((1, CL), jnp.bfloat16), ohb,
                    (((1,), (0,)), ((), ())),
                    preferred_element_type=jnp.float32)
                cnt_fix_ref[...] = counts_s - counts
                idx_ref[0, :, ci * CL:(ci + 1) * CL] = (
                    res_s[R + 1:R + 2] * 16.0
                    + res_s[R + 2:R + 3]).astype(jnp.int32)
                mask_ref[0, :, ci * CL:(ci + 1) * CL] = res_s[0:R]

            @pl.when(jnp.max(tc) <= 1.5)
            def _no_tie():
                cnt_fix_ref[...] = jnp.zeros((1, K), jnp.float32)

            counts = counts + cnt_fix_ref[...]
            counts_acc = counts if ci == 0 else counts_acc + counts
        mask_all = mask_ref[0]
        den = jnp.sum(mask_all, axis=1, keepdims=True) + 1e-6
        s_ref[pl.ds(b * R, R), :] = (msmt_ref[...] * counts_acc
                                     * (1.0 / den))

    @pl.when(b == nb - 1)
    def _final():
        num_t = jax.lax.dot_general(
            s_ref[...], cb_ref[...], (((1,), (0,)), ((), ())),
            preferred_element_type=jnp.float32)
        out_ref[...] = num_t.reshape(out_ref.shape)


def kernel(in_feas, codebook, Wr, br, cur_f=1, epoch=0):
    Bb, Ll, Cc = in_feas.shape
    Kk = codebook.shape[0]
    Rr = Wr.shape[0]
    h = int(math.sqrt(Ll))
    w = Ll // h
    cbt = codebook.T
    cbbt = codebook.astype(jnp.bfloat16).T
    idx, mask, out = pl.pallas_call(
        _sr_kernel,
        grid=(Bb + 1,),
        in_specs=[
            pl.BlockSpec((1, Ll, Cc), lambda b: (jnp.minimum(b, Bb - 1), 0, 0)),
            pl.BlockSpec((Kk, Cc), lambda b: (0, 0)),
            pl.BlockSpec((Cc, Kk), lambda b: (0, 0)),
            pl.BlockSpec((Cc, Kk), lambda b: (0, 0)),
            pl.BlockSpec((Rr, Cc), lambda b: (0, 0)),
            pl.BlockSpec((1, Rr), lambda b: (0, 0)),
        ],
        out_specs=[
            pl.BlockSpec((1, 1, Ll), lambda b: (jnp.minimum(b, Bb - 1), 0, 0)),
            pl.BlockSpec((1, Rr, Ll), lambda b: (jnp.minimum(b, Bb - 1), 0, 0)),
            pl.BlockSpec((Bb, Rr, Cc), lambda b: (0, 0, 0)),
        ],
        out_shape=[
            jax.ShapeDtypeStruct((Bb, 1, Ll), jnp.int32),
            jax.ShapeDtypeStruct((Bb, Rr, Ll), jnp.float32),
            jax.ShapeDtypeStruct((Bb, Rr, Cc), jnp.float32),
        ],
        scratch_shapes=[
            pltpu.VMEM((Rr, Kk), jnp.float32),
            pltpu.VMEM((1, Kk), jnp.float32),
            pltpu.VMEM((Rr + 8, Kk), jnp.bfloat16),
            pltpu.VMEM((Bb * Rr, Kk), jnp.float32),
            pltpu.VMEM((1, Kk), jnp.float32),
        ],
    )(in_feas, codebook, cbt, cbbt, Wr, br.reshape(1, Rr))
    enc_idx = idx.reshape(Bb, h, w)
    region_mask = mask.reshape(Bb, Rr, h, w)
    return (out, enc_idx, region_mask)


# speedup vs baseline: 1.5899x; 1.5899x over previous
import math

import jax
import jax.numpy as jnp
from jax.experimental import pallas as pl
from jax.experimental.pallas import tpu as pltpu


def _sr_kernel(z_ref, cb_ref, cbt_ref, cbbt_ref, wr_ref, br_ref,
               idx_ref, mask_ref, out_ref,
               msmt_ref, cbsq_ref, tbl_ref, s_ref):
    b = pl.program_id(0)
    nb = pl.num_programs(0)
    L = z_ref.shape[1]
    K = cb_ref.shape[0]
    R = wr_ref.shape[0]

    @pl.when(b == 0)
    def _init():
        cwr = jax.lax.dot_general(
            cb_ref[...], wr_ref[...], (((1,), (1,)), ((), ())),
            preferred_element_type=jnp.float32)
        logits = cwr + br_ref[0][None, :]
        mx = jnp.max(logits, axis=1, keepdims=True)
        e = jnp.exp(logits - mx)
        msm = e / jnp.sum(e, axis=1, keepdims=True)
        msmt = msm.T
        msmt_ref[...] = msmt
        cbsq_ref[...] = jnp.sum(cbt_ref[...] * cbt_ref[...], axis=0,
                                keepdims=True)
        tbl_ref[0:R, :] = msmt.astype(jnp.bfloat16)
        r_i = jax.lax.broadcasted_iota(jnp.int32, (8, K), 0)
        k_i = jax.lax.broadcasted_iota(jnp.int32, (8, K), 1)
        w = jnp.where(r_i == 0, 1,
                      jnp.where(r_i == 1, k_i // 16,
                                jnp.where(r_i == 2, k_i % 16, 0)))
        tbl_ref[R:R + 8, :] = w.astype(jnp.bfloat16)

    @pl.when(b < nb - 1)
    def _batch():
        z = z_ref[0]
        zsq = jnp.sum(z * z, axis=1, keepdims=True)
        zc = jax.lax.dot_general(
            z.astype(jnp.bfloat16), cbbt_ref[...],
            (((1,), (0,)), ((), ())),
            preferred_element_type=jnp.float32)
        d = zsq - 2.0 * zc + cbsq_ref[...]
        dmin = jnp.min(d, axis=1, keepdims=True)
        mtie = d == dmin
        mtb = mtie.astype(jnp.bfloat16)
        res = jax.lax.dot_general(
            tbl_ref[...], mtb, (((1,), (1,)), ((), ())),
            preferred_element_type=jnp.float32)
        counts = jax.lax.dot_general(
            jnp.ones((1, L), jnp.bfloat16), mtb, (((1,), (0,)), ((), ())),
            preferred_element_type=jnp.float32)
        tc = res[R:R + 1]

        @pl.when(jnp.max(tc) > 1.5)
        def _tie_fix():
            iota_l = jax.lax.broadcasted_iota(jnp.int32, (L, K), 1)
            strict = jnp.min(jnp.where(mtie, iota_l, K), axis=1,
                             keepdims=True)
            ohb = (iota_l == strict).astype(jnp.bfloat16)
            res_s = jax.lax.dot_general(
                tbl_ref[...], ohb, (((1,), (1,)), ((), ())),
                preferred_element_type=jnp.float32)
            counts_s = jax.lax.dot_general(
                jnp.ones((1, L), jnp.bfloat16), ohb, (((1,), (0,)), ((), ())),
                preferred_element_type=jnp.float32)
            mask_s = res_s[0:R]
            den_s = jnp.sum(mask_s, axis=1, keepdims=True) + 1e-6
            idx_ref[0] = (res_s[R + 1:R + 2] * 16.0
                          + res_s[R + 2:R + 3]).astype(jnp.int32)
            mask_ref[0] = mask_s
            s_ref[pl.ds(b * R, R), :] = (msmt_ref[...] * counts_s
                                         * (1.0 / den_s))

        @pl.when(jnp.max(tc) <= 1.5)
        def _fast():
            mask_t = res[0:R]
            den = jnp.sum(mask_t, axis=1, keepdims=True) + 1e-6
            idx_ref[0] = (res[R + 1:R + 2] * 16.0
                          + res[R + 2:R + 3]).astype(jnp.int32)
            mask_ref[0] = mask_t
            s_ref[pl.ds(b * R, R), :] = msmt_ref[...] * counts * (1.0 / den)

    @pl.when(b == nb - 1)
    def _final():
        num_t = jax.lax.dot_general(
            s_ref[...], cb_ref[...], (((1,), (0,)), ((), ())),
            preferred_element_type=jnp.float32)
        out_ref[...] = num_t.reshape(out_ref.shape)


def kernel(in_feas, codebook, Wr, br, cur_f=1, epoch=0):
    Bb, Ll, Cc = in_feas.shape
    Kk = codebook.shape[0]
    Rr = Wr.shape[0]
    h = int(math.sqrt(Ll))
    w = Ll // h
    cbt = codebook.T
    cbbt = codebook.astype(jnp.bfloat16).T
    idx, mask, out = pl.pallas_call(
        _sr_kernel,
        grid=(Bb + 1,),
        in_specs=[
            pl.BlockSpec((1, Ll, Cc), lambda b: (jnp.minimum(b, Bb - 1), 0, 0)),
            pl.BlockSpec((Kk, Cc), lambda b: (0, 0)),
            pl.BlockSpec((Cc, Kk), lambda b: (0, 0)),
            pl.BlockSpec((Cc, Kk), lambda b: (0, 0)),
            pl.BlockSpec((Rr, Cc), lambda b: (0, 0)),
            pl.BlockSpec((1, Rr), lambda b: (0, 0)),
        ],
        out_specs=[
            pl.BlockSpec((1, 1, Ll), lambda b: (jnp.minimum(b, Bb - 1), 0, 0)),
            pl.BlockSpec((1, Rr, Ll), lambda b: (jnp.minimum(b, Bb - 1), 0, 0)),
            pl.BlockSpec((Bb, Rr, Cc), lambda b: (0, 0, 0)),
        ],
        out_shape=[
            jax.ShapeDtypeStruct((Bb, 1, Ll), jnp.int32),
            jax.ShapeDtypeStruct((Bb, Rr, Ll), jnp.float32),
            jax.ShapeDtypeStruct((Bb, Rr, Cc), jnp.float32),
        ],
        scratch_shapes=[
            pltpu.VMEM((Rr, Kk), jnp.float32),
            pltpu.VMEM((1, Kk), jnp.float32),
            pltpu.VMEM((Rr + 8, Kk), jnp.bfloat16),
            pltpu.VMEM((Bb * Rr, Kk), jnp.float32),
        ],
    )(in_feas, codebook, cbt, cbbt, Wr, br.reshape(1, Rr))
    enc_idx = idx.reshape(Bb, h, w)
    region_mask = mask.reshape(Bb, Rr, h, w)
    return (out, enc_idx, region_mask)
